# baseline (speedup 1.0000x reference)
import functools

import jax
import jax.numpy as jnp
from jax import lax
from jax.experimental import pallas as pl
from jax.experimental.pallas import tpu as pltpu
from jax.experimental.pallas import tpu_sc as plsc

B = 16384
LSEQ = 200
D = 64
V = 1000000
LABELS = 1000
LANES = 16
NCOL = D // LANES

NC = 2
NS = 16
NW = NC * NS

C = 100
CPS = LSEQ // C
SEG_W = B // NW
CH_W = SEG_W * CPS
GBLK = 256
NBLK = CH_W // GBLK
RING = 8
GRPS = GBLK // RING
assert GBLK % RING == 0 and RING % CPS == 0


def _accum_chunk(buf, accs):
    def rbody(r, accs):
        out = list(accs)
        for rr in range(10):
            row = r * 10 + rr
            for cc in range(NCOL):
                out[cc] = out[cc] + buf[row, pl.ds(16 * cc, LANES)]
        return tuple(out)
    return lax.fori_loop(0, C // 10, rbody, accs)


def _sc_pool(x2d, table):
    mesh = plsc.VectorSubcoreMesh(
        core_axis_name="c", subcore_axis_name="s", num_cores=NC,
        num_subcores=NS)

    @functools.partial(
        pl.kernel,
        out_type=jax.ShapeDtypeStruct((B, D), jnp.float32),
        mesh=mesh,
        compiler_params=pltpu.CompilerParams(use_tc_tiling_on_sc=False),
        scratch_types=[
            pltpu.VMEM((GBLK, C), jnp.int32),
            [pltpu.VMEM((C, D), jnp.float32) for _ in range(RING)],
            pltpu.VMEM((SEG_W, D), jnp.float32),
            [pltpu.SemaphoreType.DMA for _ in range(RING)],
        ],
    )
    def k(x_hbm, tab_hbm, out_hbm, idx_v, bufs, acc_v, sems):
        cid = lax.axis_index("c")
        sid = lax.axis_index("s")
        wid = sid * NC + cid
        chunk_base = wid * CH_W
        seg_base = wid * SEG_W
        scale = jnp.full((LANES,), 1.0 / LSEQ, jnp.float32)
        zero = jnp.zeros((LANES,), jnp.float32)

        def start(j, t):
            pltpu.async_copy(tab_hbm.at[idx_v.at[j]], bufs[t], sems[t])

        def wait(j, t):
            pltpu.make_async_copy(
                tab_hbm.at[idx_v.at[j]], bufs[t], sems[t]).wait()

        def block_body(blk, _):
            pltpu.sync_copy(
                x_hbm.at[pl.ds(chunk_base + blk * GBLK, GBLK)], idx_v)
            for t in range(RING):
                start(t, t)

            seg0 = blk * (GBLK // CPS)

            def drain(g, issue_next):
                accs = None
                for t in range(RING):
                    j = g * RING + t
                    wait(j, t)
                    if t % CPS == 0:
                        accs = _accum_chunk(bufs[t], (zero,) * NCOL)
                    else:
                        accs = _accum_chunk(bufs[t], accs)
                        seg = seg0 + g * (RING // CPS) + t // CPS
                        for cc in range(NCOL):
                            acc_v[seg, pl.ds(16 * cc, LANES)] = (
                                accs[cc] * scale)
                    if issue_next:
                        start(j + RING, t)

            def grp_body(g, _):
                drain(g, True)
                return 0

            lax.fori_loop(0, GRPS - 1, grp_body, 0)
            drain(GRPS - 1, False)
            return 0

        lax.fori_loop(0, NBLK, block_body, 0)
        pltpu.sync_copy(acc_v, out_hbm.at[pl.ds(seg_base, SEG_W)])

    return k(x2d, table)


def _tc_head(pooled, W, b2d):
    TB = 1024

    def mm(w_ref, p_ref, b_ref, o_ref):
        acc = lax.dot_general(
            w_ref[...], p_ref[...], (((1,), (1,)), ((), ())),
            preferred_element_type=jnp.float32)
        o_ref[...] = acc + b_ref[...]

    return pl.pallas_call(
        mm,
        grid=(B // TB,),
        in_specs=[
            pl.BlockSpec((LABELS, D), lambda i: (0, 0)),
            pl.BlockSpec((TB, D), lambda i: (i, 0)),
            pl.BlockSpec((LABELS, 1), lambda i: (0, 0)),
        ],
        out_specs=pl.BlockSpec((LABELS, TB), lambda i: (0, i)),
        out_shape=jax.ShapeDtypeStruct((LABELS, B), jnp.float32),
    )(W, pooled, b2d)


def kernel(x, table, W, b):
    x2d = x.reshape(B * LSEQ // C, C)
    pooled = _sc_pool(x2d, table)
    return _tc_head(pooled, W, b.reshape(LABELS, 1)).T
